# baseline (speedup 1.0000x reference)
import functools

import jax
import jax.numpy as jnp
from jax import lax
from jax.experimental import pallas as pl
from jax.experimental.pallas import tpu as pltpu
from jax.experimental.pallas import tpu_sc as plsc


def _make_sc_kernel(B, D, NC, NS, L):
    NW = NC * NS
    b_per_w = B // NW
    CH = 128
    n_ch = b_per_w // CH

    mesh = plsc.VectorSubcoreMesh(
        core_axis_name="c", subcore_axis_name="s",
        num_cores=NC, num_subcores=NS)

    @functools.partial(
        pl.kernel,
        mesh=mesh,
        compiler_params=pltpu.CompilerParams(use_tc_tiling_on_sc=False),
        out_type=jax.ShapeDtypeStruct((NW, L), jnp.float32),
        scratch_types=[
            pltpu.VMEM((n_ch, CH), jnp.int32),
            pltpu.VMEM((D, b_per_w), jnp.float32),
            pltpu.VMEM((D, b_per_w), jnp.float32),
            pltpu.VMEM((L,), jnp.float32),
            pltpu.SemaphoreType.DMA,
        ],
    )
    def sc_kernel(tab_t_hbm, feat_t_hbm, lab_hbm, out_hbm,
                  idx_v, rows_v, feat_v, acc_v, sem):
        wid = lax.axis_index("s") * NC + lax.axis_index("c")
        pltpu.sync_copy(lab_hbm.at[wid], idx_v)
        copies = [
            pltpu.make_async_copy(
                tab_t_hbm.at[c].at[idx_v.at[j]],
                rows_v.at[c, pl.ds(j * CH, CH)],
                sem)
            for c in range(D)
            for j in range(n_ch)
        ]
        for cp in copies:
            cp.start()
        pltpu.sync_copy(feat_t_hbm.at[:, pl.ds(wid * b_per_w, b_per_w)],
                        feat_v)
        for cp in copies:
            cp.wait()

        acc = jnp.zeros((L,), jnp.float32)
        for c in range(D):
            def body(i, a, c=c):
                return (a + rows_v[c, pl.ds(i * L, L)]
                        * feat_v[c, pl.ds(i * L, L)])
            acc = lax.fori_loop(0, b_per_w // L, body, acc)
        acc_v[...] = acc
        pltpu.sync_copy(acc_v, out_hbm.at[wid])

    return sc_kernel


def kernel(centres, features, labels):
    B, D = features.shape
    info = plsc.get_sparse_core_info()
    NC, NS, L = info.num_cores, info.num_subcores, info.num_lanes
    NW = NC * NS
    b_per_w = B // NW
    lab = labels.astype(jnp.int32).reshape(NW, b_per_w // 128, 128)
    tab_t = centres.T
    feat_t = features.T
    partials = _make_sc_kernel(B, D, NC, NS, L)(tab_t, feat_t, lab)
    return jnp.sum(partials)


# speedup vs baseline: 2.9944x; 2.9944x over previous
import functools

import jax
import jax.numpy as jnp
from jax import lax
from jax.experimental import pallas as pl
from jax.experimental.pallas import tpu as pltpu
from jax.experimental.pallas import tpu_sc as plsc


def _make_sc_kernel(B, D, NC, NS, L, V):
    NW = NC * NS
    b_per_w = B // NW
    CH = 128
    n_ch = b_per_w // CH
    HALF = V // 2

    mesh = plsc.VectorSubcoreMesh(
        core_axis_name="c", subcore_axis_name="s",
        num_cores=NC, num_subcores=NS)

    @functools.partial(
        pl.kernel,
        mesh=mesh,
        compiler_params=pltpu.CompilerParams(use_tc_tiling_on_sc=False),
        out_type=jax.ShapeDtypeStruct((NW, L), jnp.float32),
        scratch_types=[
            pltpu.VMEM((n_ch, CH), jnp.int32),
            pltpu.VMEM((n_ch, CH), jnp.int32),
            pltpu.VMEM((n_ch, CH), jnp.int32),
            pltpu.VMEM((b_per_w, D), jnp.float32),
            pltpu.VMEM((b_per_w, D), jnp.float32),
            pltpu.VMEM((b_per_w, D), jnp.float32),
            pltpu.VMEM((L,), jnp.float32),
            pltpu.SemaphoreType.DMA,
        ],
    )
    def sc_kernel(tab_a_hbm, tab_b_hbm, feat_hbm, ia_hbm, ib_hbm, lab_hbm,
                  out_hbm, ia_v, ib_v, lab_v, rows_a, rows_b, feat_v, acc_v,
                  sem):
        wid = lax.axis_index("s") * NC + lax.axis_index("c")
        pltpu.sync_copy(ia_hbm.at[wid], ia_v)
        pltpu.sync_copy(ib_hbm.at[wid], ib_v)
        pltpu.sync_copy(lab_hbm.at[wid], lab_v)
        copies = [
            pltpu.make_async_copy(
                tab.at[idx.at[j]],
                rows.at[pl.ds(j * CH, CH)],
                sem)
            for tab, idx, rows in ((tab_a_hbm, ia_v, rows_a),
                                   (tab_b_hbm, ib_v, rows_b))
            for j in range(n_ch)
        ]
        for cp in copies:
            cp.start()
        pltpu.sync_copy(feat_hbm.at[wid], feat_v)
        for cp in copies:
            cp.wait()

        def body(i, acc):
            k, m = i // (CH // L), i % (CH // L)
            lab16 = lab_v[k, pl.ds(m * L, L)]
            in_b0 = lab16 >= HALF
            for h in range(2):
                sel = jnp.where(in_b0, rows_b[i, pl.ds(h * L, L)],
                                rows_a[i, pl.ds(h * L, L)])
                acc = acc + sel * feat_v[i, pl.ds(h * L, L)]
            return acc

        acc = lax.fori_loop(0, b_per_w, body,
                            jnp.zeros((L,), jnp.float32))
        acc_v[...] = acc
        pltpu.sync_copy(acc_v, out_hbm.at[wid])

    return sc_kernel


def kernel(centres, features, labels):
    B, D = features.shape
    V = centres.shape[0]
    info = plsc.get_sparse_core_info()
    NC, NS, L = info.num_cores, info.num_subcores, info.num_lanes
    NW = NC * NS
    b_per_w = B // NW
    half = V // 2
    lab32 = labels.astype(jnp.int32)
    sh = (NW, b_per_w // 128, 128)
    idx_a = jnp.minimum(lab32, half - 1).reshape(sh)
    idx_b = jnp.maximum(lab32 - half, 0).reshape(sh)
    lab3d = lab32.reshape(sh)
    feat = features.reshape(NW, b_per_w, D)
    tab_a = centres[:half]
    tab_b = centres[half:]
    partials = _make_sc_kernel(B, D, NC, NS, L, V)(
        tab_a, tab_b, feat, idx_a, idx_b, lab3d)
    return jnp.sum(partials)


# speedup vs baseline: 4.9810x; 1.6635x over previous
import functools

import jax
import jax.numpy as jnp
from jax import lax
from jax.experimental import pallas as pl
from jax.experimental.pallas import tpu as pltpu
from jax.experimental.pallas import tpu_sc as plsc


def _make_sc_kernel(B, D, NC, NS, L):
    NW = NC * NS
    b_per_w = B // NW
    CH = 128
    n_ch = b_per_w // CH

    mesh = plsc.VectorSubcoreMesh(
        core_axis_name="c", subcore_axis_name="s",
        num_cores=NC, num_subcores=NS)

    @functools.partial(
        pl.kernel,
        mesh=mesh,
        compiler_params=pltpu.CompilerParams(use_tc_tiling_on_sc=False),
        out_type=jax.ShapeDtypeStruct((NW, L), jnp.float32),
        scratch_types=[
            pltpu.VMEM((n_ch, CH), jnp.int32),
            pltpu.VMEM((b_per_w, D), jnp.float32),
            pltpu.VMEM((b_per_w, D), jnp.float32),
            pltpu.VMEM((L,), jnp.float32),
            pltpu.SemaphoreType.DMA,
        ],
    )
    def sc_kernel(centres_hbm, feat_hbm, lab_hbm, out_hbm,
                  idx_v, rows_v, feat_v, acc_v, sem):
        wid = lax.axis_index("s") * NC + lax.axis_index("c")
        pltpu.sync_copy(lab_hbm.at[wid], idx_v)
        copies = [
            pltpu.make_async_copy(
                centres_hbm.at[idx_v.at[j]],
                rows_v.at[pl.ds(j * CH, CH)],
                sem)
            for j in range(n_ch)
        ]
        for c in copies:
            c.start()
        pltpu.sync_copy(feat_hbm.at[wid], feat_v)
        for c in copies:
            c.wait()

        def body(i, acc):
            a0 = rows_v[i, pl.ds(0, L)] * feat_v[i, pl.ds(0, L)]
            a1 = rows_v[i, pl.ds(L, L)] * feat_v[i, pl.ds(L, L)]
            return acc + a0 + a1

        acc = lax.fori_loop(0, b_per_w, body,
                            jnp.zeros((L,), jnp.float32))
        acc_v[...] = acc
        pltpu.sync_copy(acc_v, out_hbm.at[wid])

    return sc_kernel


def kernel(centres, features, labels):
    B, D = features.shape
    info = plsc.get_sparse_core_info()
    NC, NS, L = info.num_cores, info.num_subcores, info.num_lanes
    NW = NC * NS
    b_per_w = B // NW
    lab = labels.astype(jnp.int32).reshape(NW, b_per_w // 128, 128)
    feat = features.reshape(NW, b_per_w, D)
    partials = _make_sc_kernel(B, D, NC, NS, L)(centres, feat, lab)
    return jnp.sum(partials)
